# baseline (speedup 1.0000x reference)
import functools

import jax
import jax.numpy as jnp
from jax import lax
from jax.experimental import pallas as pl
from jax.experimental.pallas import tpu as pltpu
from jax.experimental.pallas import tpu_sc as plsc

N_REAL = 10000
D = 128
DQ = D // 4
E_REAL = 320000

NC = 2
NS = 16
NW = NC * NS
CHUNK = 128
N_CHUNKS = 2560
E_P = N_CHUNKS * CHUNK
CHUNKS_PER_W = N_CHUNKS // NW
CHUNKS_PER_T = N_CHUNKS // NS
N_P = 10240
ROWS_PER_TILE = N_P // NS
BLK = 512

_mesh = plsc.VectorSubcoreMesh(core_axis_name="c", subcore_axis_name="s")


def _mm_nt(a, b):
    return lax.dot_general(a, b, (((1,), (1,)), ((), ())),
                           preferred_element_type=jnp.float32)


def _hw_body(x_ref, w1_ref, b1_ref, wg_ref, out_ref):
    h = jnp.maximum(_mm_nt(x_ref[...], w1_ref[...]) + b1_ref[...], 0.0)
    out_ref[...] = _mm_nt(h, wg_ref[...])


def _tc_hw(xp, W1, b1_2d, Wg):
    return pl.pallas_call(
        _hw_body,
        grid=(N_P // BLK,),
        in_specs=[
            pl.BlockSpec((BLK, D), lambda i: (i, 0)),
            pl.BlockSpec((D, D), lambda i: (0, 0)),
            pl.BlockSpec((1, D), lambda i: (0, 0)),
            pl.BlockSpec((D, D), lambda i: (0, 0)),
        ],
        out_specs=pl.BlockSpec((BLK, D), lambda i: (i, 0)),
        out_shape=jax.ShapeDtypeStruct((N_P, D), jnp.float32),
    )(xp, W1, b1_2d, Wg)


@functools.partial(
    pl.kernel,
    out_type=jax.ShapeDtypeStruct((NC, N_P), jnp.float32),
    mesh=_mesh,
    scratch_types=[
        pltpu.VMEM((CHUNKS_PER_W, CHUNK), jnp.int32),
        pltpu.VMEM((CHUNK,), jnp.float32),
        pltpu.VMEM((ROWS_PER_TILE,), jnp.float32),
        pltpu.VMEM_SHARED((N_P,), jnp.float32),
    ],
)
def _sc_degree(dst_hbm, out_hbm, idx_v, ones_v, buf_v, acc_sh):
    c = lax.axis_index("c")
    s = lax.axis_index("s")
    wid = s * NC + c

    def fill16(i, _):
        ones_v[pl.ds(i * 16, 16)] = jnp.ones((16,), jnp.float32)
        return 0
    lax.fori_loop(0, CHUNK // 16, fill16, 0)

    def zero16(i, _):
        buf_v[pl.ds(i * 16, 16)] = jnp.zeros((16,), jnp.float32)
        return 0
    lax.fori_loop(0, ROWS_PER_TILE // 16, zero16, 0)

    pltpu.sync_copy(buf_v, acc_sh.at[pl.ds(s * ROWS_PER_TILE, ROWS_PER_TILE)])
    plsc.subcore_barrier()

    pltpu.sync_copy(dst_hbm.at[pl.ds(wid * CHUNKS_PER_W, CHUNKS_PER_W)], idx_v)

    def body(j, _):
        pltpu.sync_copy(ones_v, acc_sh.at[idx_v.at[j]], add=True)
        return 0
    lax.fori_loop(0, CHUNKS_PER_W, body, 0)

    plsc.subcore_barrier()
    pltpu.sync_copy(acc_sh.at[pl.ds(s * ROWS_PER_TILE, ROWS_PER_TILE)], buf_v)
    pltpu.sync_copy(buf_v, out_hbm.at[c, pl.ds(s * ROWS_PER_TILE, ROWS_PER_TILE)])


def _scale_body(hw_ref, d0_ref, d1_ref, g0_ref, g1_ref, g2_ref, g3_ref,
                dinv_ref):
    deg = d0_ref[...] + d1_ref[...] + 1.0
    dinv = lax.rsqrt(deg)
    dinv_ref[...] = dinv
    g = hw_ref[...] * dinv
    g0_ref[...] = g[:, 0 * DQ:1 * DQ]
    g1_ref[...] = g[:, 1 * DQ:2 * DQ]
    g2_ref[...] = g[:, 2 * DQ:3 * DQ]
    g3_ref[...] = g[:, 3 * DQ:4 * DQ]


def _tc_scale(hw, deg0, deg1):
    return pl.pallas_call(
        _scale_body,
        grid=(N_P // BLK,),
        in_specs=[
            pl.BlockSpec((BLK, D), lambda i: (i, 0)),
            pl.BlockSpec((BLK, 1), lambda i: (i, 0)),
            pl.BlockSpec((BLK, 1), lambda i: (i, 0)),
        ],
        out_specs=[
            pl.BlockSpec((BLK, DQ), lambda i: (i, 0)),
            pl.BlockSpec((BLK, DQ), lambda i: (i, 0)),
            pl.BlockSpec((BLK, DQ), lambda i: (i, 0)),
            pl.BlockSpec((BLK, DQ), lambda i: (i, 0)),
            pl.BlockSpec((BLK, 1), lambda i: (i, 0)),
        ],
        out_shape=[
            jax.ShapeDtypeStruct((N_P, DQ), jnp.float32),
            jax.ShapeDtypeStruct((N_P, DQ), jnp.float32),
            jax.ShapeDtypeStruct((N_P, DQ), jnp.float32),
            jax.ShapeDtypeStruct((N_P, DQ), jnp.float32),
            jax.ShapeDtypeStruct((N_P, 1), jnp.float32),
        ],
    )(hw, deg0, deg1)


@functools.partial(
    pl.kernel,
    out_type=[jax.ShapeDtypeStruct((N_P, DQ), jnp.float32)] * 4,
    mesh=_mesh,
    scratch_types=[
        pltpu.VMEM((CHUNKS_PER_T, CHUNK), jnp.int32),
        pltpu.VMEM((CHUNKS_PER_T, CHUNK), jnp.int32),
        pltpu.VMEM((4, CHUNK, DQ), jnp.float32),
        pltpu.VMEM((CHUNK, DQ), jnp.float32),
        pltpu.VMEM_SHARED((N_P, DQ), jnp.float32),
        pltpu.VMEM_SHARED((N_P, DQ), jnp.float32),
        [pltpu.SemaphoreType.DMA] * 4,
        [pltpu.SemaphoreType.DMA] * 4,
    ],
    compiler_params=pltpu.CompilerParams(use_tc_tiling_on_sc=False),
)
def _sc_scatter(src_hbm, dst_hbm, g0_hbm, g1_hbm, g2_hbm, g3_hbm,
                a0_hbm, a1_hbm, a2_hbm, a3_hbm,
                src_v, dst_v, ring, zbuf, acc_sh, tab_sh, gsems, ssems):
    c = lax.axis_index("c")
    s = lax.axis_index("s")
    rows = pl.ds(s * ROWS_PER_TILE, ROWS_PER_TILE)

    def zero_row(i, _):
        zbuf[i // (DQ // 16), pl.ds((i % (DQ // 16)) * 16, 16)] = (
            jnp.zeros((16,), jnp.float32))
        return 0
    lax.fori_loop(0, CHUNK * (DQ // 16), zero_row, 0)

    pltpu.sync_copy(src_hbm.at[pl.ds(s * CHUNKS_PER_T, CHUNKS_PER_T)], src_v)
    pltpu.sync_copy(dst_hbm.at[pl.ds(s * CHUNKS_PER_T, CHUNKS_PER_T)], dst_v)

    def do_round(g_hbm, out_hbm):
        pltpu.sync_copy(g_hbm.at[rows], tab_sh.at[rows])

        def zero_acc(k, _):
            pltpu.sync_copy(zbuf,
                            acc_sh.at[pl.ds(s * ROWS_PER_TILE + k * CHUNK, CHUNK)])
            return 0
        lax.fori_loop(0, ROWS_PER_TILE // CHUNK, zero_acc, 0)
        plsc.subcore_barrier()

        for k in range(3):
            pltpu.async_copy(tab_sh.at[src_v.at[k]], ring.at[k], gsems[k])

        def body(jj, _):
            for k in range(4):
                j = jj * 4 + k
                m = (k + 3) % 4
                pltpu.make_async_copy(tab_sh.at[src_v.at[j]], ring.at[k],
                                      gsems[k]).wait()

                @pl.when(jnp.logical_and(j >= 1, j + 3 < CHUNKS_PER_T))
                def _():
                    pltpu.make_async_copy(ring.at[m],
                                          acc_sh.at[dst_v.at[j]],
                                          ssems[m]).wait()

                @pl.when(j + 3 < CHUNKS_PER_T)
                def _():
                    pltpu.async_copy(tab_sh.at[src_v.at[j + 3]], ring.at[m],
                                     gsems[m])

                pltpu.async_copy(ring.at[k], acc_sh.at[dst_v.at[j]],
                                 ssems[k], add=True)
            return 0
        lax.fori_loop(0, CHUNKS_PER_T // 4, body, 0)

        for k in range(4):
            pltpu.make_async_copy(ring.at[k], acc_sh.at[dst_v.at[0]],
                                  ssems[k]).wait()

        plsc.subcore_barrier()

        def wo(k, _):
            off = s * ROWS_PER_TILE + k * CHUNK
            pltpu.sync_copy(acc_sh.at[pl.ds(off, CHUNK)], ring.at[0])
            pltpu.sync_copy(ring.at[0], out_hbm.at[pl.ds(off, CHUNK)])
            return 0
        lax.fori_loop(0, ROWS_PER_TILE // CHUNK, wo, 0)
        plsc.subcore_barrier()

    @pl.when(c == 0)
    def _():
        do_round(g0_hbm, a0_hbm)
        do_round(g1_hbm, a1_hbm)

    @pl.when(c == 1)
    def _():
        do_round(g2_hbm, a2_hbm)
        do_round(g3_hbm, a3_hbm)


def _out_body(a0_ref, a1_ref, a2_ref, a3_ref, g0_ref, g1_ref, g2_ref, g3_ref,
              dinv_ref, bg_ref, w2_ref, b2_ref, o_ref):
    dinv = dinv_ref[...]
    conv = jnp.concatenate(
        [(a0_ref[...] + g0_ref[...]) * dinv,
         (a1_ref[...] + g1_ref[...]) * dinv,
         (a2_ref[...] + g2_ref[...]) * dinv,
         (a3_ref[...] + g3_ref[...]) * dinv], axis=1) + bg_ref[...]
    h2 = jnp.maximum(conv, 0.0)
    o_ref[...] = _mm_nt(h2, w2_ref[...]) + b2_ref[...]


def _tc_out(a_slabs, g_slabs, dinv, bg_2d, W2, b2_2d):
    slab_spec = pl.BlockSpec((BLK, DQ), lambda i: (i, 0))
    return pl.pallas_call(
        _out_body,
        grid=(N_P // BLK,),
        in_specs=[slab_spec] * 8 + [
            pl.BlockSpec((BLK, 1), lambda i: (i, 0)),
            pl.BlockSpec((1, D), lambda i: (0, 0)),
            pl.BlockSpec((D, D), lambda i: (0, 0)),
            pl.BlockSpec((1, D), lambda i: (0, 0)),
        ],
        out_specs=pl.BlockSpec((BLK, D), lambda i: (i, 0)),
        out_shape=jax.ShapeDtypeStruct((N_P, D), jnp.float32),
    )(*a_slabs, *g_slabs, dinv, bg_2d, W2, b2_2d)


def kernel(x, edge_index, W1, b1, Wg, bg, W2, b2):
    ei = edge_index.astype(jnp.int32)
    n_pad_e = E_P - E_REAL
    pad_idx = N_REAL + (jnp.arange(n_pad_e, dtype=jnp.int32) % (N_P - N_REAL))
    srcp = jnp.concatenate([ei[0], pad_idx]).reshape(N_CHUNKS, CHUNK)
    dstp = jnp.concatenate([ei[1], pad_idx]).reshape(N_CHUNKS, CHUNK)

    xp = jnp.pad(x, ((0, N_P - N_REAL), (0, 0)))
    b1_2d = b1.reshape(1, D)
    bg_2d = bg.reshape(1, D)
    b2_2d = b2.reshape(1, D)

    hw = _tc_hw(xp, W1, b1_2d, Wg)

    deg_partials = _sc_degree(dstp)
    deg0 = deg_partials[0].reshape(N_P, 1)
    deg1 = deg_partials[1].reshape(N_P, 1)

    g0, g1, g2, g3, dinv = _tc_scale(hw, deg0, deg1)

    a0, a1, a2, a3 = _sc_scatter(srcp, dstp, g0, g1, g2, g3)

    out = _tc_out((a0, a1, a2, a3), (g0, g1, g2, g3), dinv, bg_2d, W2, b2_2d)
    return out[:N_REAL]


# speedup vs baseline: 1.3703x; 1.3703x over previous
import functools

import jax
import jax.numpy as jnp
from jax import lax
from jax.experimental import pallas as pl
from jax.experimental.pallas import tpu as pltpu
from jax.experimental.pallas import tpu_sc as plsc

N_REAL = 10000
D = 128
DH = D // 2
E_REAL = 320000

NC = 2
NS = 16
NW = NC * NS
CHUNK = 128
N_CHUNKS = 2560
E_P = N_CHUNKS * CHUNK
CHUNKS_PER_W = N_CHUNKS // NW
CHUNKS_PER_T = N_CHUNKS // NS
N_P = 10240
ROWS_PER_TILE = N_P // NS
BLK = 512

_mesh = plsc.VectorSubcoreMesh(core_axis_name="c", subcore_axis_name="s")


def _mm_nt(a, b):
    return lax.dot_general(a, b, (((1,), (1,)), ((), ())),
                           preferred_element_type=jnp.float32)


@functools.partial(
    pl.kernel,
    out_type=jax.ShapeDtypeStruct((NC, N_P), jnp.float32),
    mesh=_mesh,
    scratch_types=[
        pltpu.VMEM((CHUNKS_PER_W, CHUNK), jnp.int32),
        pltpu.VMEM((CHUNK,), jnp.float32),
        pltpu.VMEM((ROWS_PER_TILE,), jnp.float32),
        pltpu.VMEM_SHARED((N_P,), jnp.float32),
    ],
)
def _sc_degree(dst_hbm, out_hbm, idx_v, ones_v, buf_v, acc_sh):
    c = lax.axis_index("c")
    s = lax.axis_index("s")
    wid = s * NC + c

    def fill16(i, _):
        ones_v[pl.ds(i * 16, 16)] = jnp.ones((16,), jnp.float32)
        return 0
    lax.fori_loop(0, CHUNK // 16, fill16, 0)

    def zero16(i, _):
        buf_v[pl.ds(i * 16, 16)] = jnp.zeros((16,), jnp.float32)
        return 0
    lax.fori_loop(0, ROWS_PER_TILE // 16, zero16, 0)

    pltpu.sync_copy(buf_v, acc_sh.at[pl.ds(s * ROWS_PER_TILE, ROWS_PER_TILE)])
    plsc.subcore_barrier()

    pltpu.sync_copy(dst_hbm.at[pl.ds(wid * CHUNKS_PER_W, CHUNKS_PER_W)], idx_v)

    def body(j, _):
        pltpu.sync_copy(ones_v, acc_sh.at[idx_v.at[j]], add=True)
        return 0
    lax.fori_loop(0, CHUNKS_PER_W, body, 0)

    plsc.subcore_barrier()
    rows = pl.ds(s * ROWS_PER_TILE, ROWS_PER_TILE)
    pltpu.sync_copy(acc_sh.at[rows], buf_v)
    pltpu.sync_copy(buf_v, out_hbm.at[c, rows])


def _fused_body(x_ref, w1_ref, b1_ref, wg_ref, d0_ref, d1_ref,
                glo_ref, ghi_ref, dinv_ref):
    h = jnp.maximum(_mm_nt(x_ref[...], w1_ref[...]) + b1_ref[...], 0.0)
    hw = _mm_nt(h, wg_ref[...])
    deg = d0_ref[...] + d1_ref[...] + 1.0
    dinv = lax.rsqrt(deg)
    dinv_ref[...] = dinv
    g = hw * dinv
    glo_ref[...] = g[:, :DH]
    ghi_ref[...] = g[:, DH:]


def _tc_fused(xp, W1, b1_2d, Wg, deg0, deg1):
    return pl.pallas_call(
        _fused_body,
        grid=(N_P // BLK,),
        in_specs=[
            pl.BlockSpec((BLK, D), lambda i: (i, 0)),
            pl.BlockSpec((D, D), lambda i: (0, 0)),
            pl.BlockSpec((1, D), lambda i: (0, 0)),
            pl.BlockSpec((D, D), lambda i: (0, 0)),
            pl.BlockSpec((BLK, 1), lambda i: (i, 0)),
            pl.BlockSpec((BLK, 1), lambda i: (i, 0)),
        ],
        out_specs=[
            pl.BlockSpec((BLK, DH), lambda i: (i, 0)),
            pl.BlockSpec((BLK, DH), lambda i: (i, 0)),
            pl.BlockSpec((BLK, 1), lambda i: (i, 0)),
        ],
        out_shape=[
            jax.ShapeDtypeStruct((N_P, DH), jnp.float32),
            jax.ShapeDtypeStruct((N_P, DH), jnp.float32),
            jax.ShapeDtypeStruct((N_P, 1), jnp.float32),
        ],
    )(xp, W1, b1_2d, Wg, deg0, deg1)


@functools.partial(
    pl.kernel,
    out_type=[
        jax.ShapeDtypeStruct((N_P, DH), jnp.float32),
        jax.ShapeDtypeStruct((N_P, DH), jnp.float32),
    ],
    mesh=_mesh,
    scratch_types=[
        pltpu.VMEM((CHUNKS_PER_T, CHUNK), jnp.int32),
        pltpu.VMEM((CHUNKS_PER_T, CHUNK), jnp.int32),
        pltpu.VMEM((4, CHUNK, DH), jnp.float32),
        pltpu.VMEM_SHARED((N_P, DH), jnp.float32),
        [pltpu.SemaphoreType.DMA] * 4,
        [pltpu.SemaphoreType.DMA] * 4,
    ],
    compiler_params=pltpu.CompilerParams(use_tc_tiling_on_sc=False),
)
def _sc_scatter(src_hbm, dst_hbm, glo_hbm, ghi_hbm, outlo_hbm, outhi_hbm,
                src_v, dst_v, ring, acc_sh, gsems, ssems):
    c = lax.axis_index("c")
    s = lax.axis_index("s")

    def zero_row(i, _):
        ring[0, i // (DH // 16), pl.ds((i % (DH // 16)) * 16, 16)] = (
            jnp.zeros((16,), jnp.float32))
        return 0
    lax.fori_loop(0, CHUNK * (DH // 16), zero_row, 0)

    def zero_acc(k, _):
        pltpu.sync_copy(ring.at[0],
                        acc_sh.at[pl.ds(s * ROWS_PER_TILE + k * CHUNK, CHUNK)])
        return 0
    lax.fori_loop(0, ROWS_PER_TILE // CHUNK, zero_acc, 0)
    plsc.subcore_barrier()

    pltpu.sync_copy(src_hbm.at[pl.ds(s * CHUNKS_PER_T, CHUNKS_PER_T)], src_v)
    pltpu.sync_copy(dst_hbm.at[pl.ds(s * CHUNKS_PER_T, CHUNKS_PER_T)], dst_v)

    def gather(g_hbm):
        for k in range(3):
            pltpu.async_copy(g_hbm.at[src_v.at[k]], ring.at[k], gsems[k])

        def body(jj, _):
            for k in range(4):
                j = jj * 4 + k
                m = (k + 3) % 4
                pltpu.make_async_copy(g_hbm.at[src_v.at[j]], ring.at[k],
                                      gsems[k]).wait()

                @pl.when(jnp.logical_and(j >= 1, j + 3 < CHUNKS_PER_T))
                def _():
                    pltpu.make_async_copy(ring.at[m],
                                          acc_sh.at[dst_v.at[j]],
                                          ssems[m]).wait()

                @pl.when(j + 3 < CHUNKS_PER_T)
                def _():
                    pltpu.async_copy(g_hbm.at[src_v.at[j + 3]], ring.at[m],
                                     gsems[m])

                pltpu.async_copy(ring.at[k], acc_sh.at[dst_v.at[j]],
                                 ssems[k], add=True)
            return 0
        lax.fori_loop(0, CHUNKS_PER_T // 4, body, 0)

        for k in range(4):
            pltpu.make_async_copy(ring.at[k], acc_sh.at[dst_v.at[0]],
                                  ssems[k]).wait()

    @pl.when(c == 0)
    def _():
        gather(glo_hbm)

    @pl.when(c == 1)
    def _():
        gather(ghi_hbm)

    plsc.subcore_barrier()

    def writeout(out_hbm):
        def wo(k, _):
            off = s * ROWS_PER_TILE + k * CHUNK
            pltpu.sync_copy(acc_sh.at[pl.ds(off, CHUNK)], ring.at[0])
            pltpu.sync_copy(ring.at[0], out_hbm.at[pl.ds(off, CHUNK)])
            return 0
        lax.fori_loop(0, ROWS_PER_TILE // CHUNK, wo, 0)

    @pl.when(c == 0)
    def _():
        writeout(outlo_hbm)

    @pl.when(c == 1)
    def _():
        writeout(outhi_hbm)


def _out_body(alo_ref, ahi_ref, glo_ref, ghi_ref, dinv_ref, bg_ref,
              w2_ref, b2_ref, o_ref):
    dinv = dinv_ref[...]
    conv = jnp.concatenate(
        [(alo_ref[...] + glo_ref[...]) * dinv,
         (ahi_ref[...] + ghi_ref[...]) * dinv], axis=1) + bg_ref[...]
    h2 = jnp.maximum(conv, 0.0)
    o_ref[...] = _mm_nt(h2, w2_ref[...]) + b2_ref[...]


def _tc_out(alo, ahi, glo, ghi, dinv, bg_2d, W2, b2_2d):
    return pl.pallas_call(
        _out_body,
        grid=(N_P // BLK,),
        in_specs=[
            pl.BlockSpec((BLK, DH), lambda i: (i, 0)),
            pl.BlockSpec((BLK, DH), lambda i: (i, 0)),
            pl.BlockSpec((BLK, DH), lambda i: (i, 0)),
            pl.BlockSpec((BLK, DH), lambda i: (i, 0)),
            pl.BlockSpec((BLK, 1), lambda i: (i, 0)),
            pl.BlockSpec((1, D), lambda i: (0, 0)),
            pl.BlockSpec((D, D), lambda i: (0, 0)),
            pl.BlockSpec((1, D), lambda i: (0, 0)),
        ],
        out_specs=pl.BlockSpec((BLK, D), lambda i: (i, 0)),
        out_shape=jax.ShapeDtypeStruct((N_P, D), jnp.float32),
    )(alo, ahi, glo, ghi, dinv, bg_2d, W2, b2_2d)


def kernel(x, edge_index, W1, b1, Wg, bg, W2, b2):
    ei = edge_index.astype(jnp.int32)
    n_pad_e = E_P - E_REAL
    pad_idx = N_REAL + (jnp.arange(n_pad_e, dtype=jnp.int32) % (N_P - N_REAL))
    srcp = jnp.concatenate([ei[0], pad_idx]).reshape(N_CHUNKS, CHUNK)
    dstp = jnp.concatenate([ei[1], pad_idx]).reshape(N_CHUNKS, CHUNK)

    xp = jnp.pad(x, ((0, N_P - N_REAL), (0, 0)))
    b1_2d = b1.reshape(1, D)
    bg_2d = bg.reshape(1, D)
    b2_2d = b2.reshape(1, D)

    deg_partials = _sc_degree(dstp)
    deg0 = deg_partials[0].reshape(N_P, 1)
    deg1 = deg_partials[1].reshape(N_P, 1)

    glo, ghi, dinv = _tc_fused(xp, W1, b1_2d, Wg, deg0, deg1)

    alo, ahi = _sc_scatter(srcp, dstp, glo, ghi)

    out = _tc_out(alo, ahi, glo, ghi, dinv, bg_2d, W2, b2_2d)
    return out[:N_REAL]


# speedup vs baseline: 1.6073x; 1.1729x over previous
import functools

import jax
import jax.numpy as jnp
from jax import lax
from jax.experimental import pallas as pl
from jax.experimental.pallas import tpu as pltpu
from jax.experimental.pallas import tpu_sc as plsc

N_REAL = 10000
D = 128
DH = D // 2
E_REAL = 320000

NC = 2
NS = 16
NW = NC * NS
CHUNK = 128
N_CHUNKS = 2560
E_P = N_CHUNKS * CHUNK
CHUNKS_PER_W = N_CHUNKS // NW
CHUNKS_PER_T = N_CHUNKS // NS
N_P = 10240
ROWS_PER_TILE = N_P // NS
BLK = 512

_mesh = plsc.VectorSubcoreMesh(core_axis_name="c", subcore_axis_name="s")


def _mm_nt(a, b):
    return lax.dot_general(a, b, (((1,), (1,)), ((), ())),
                           preferred_element_type=jnp.float32)


@functools.partial(
    pl.kernel,
    out_type=jax.ShapeDtypeStruct((NC, N_P), jnp.float32),
    mesh=_mesh,
    scratch_types=[
        pltpu.VMEM((CHUNKS_PER_W, CHUNK), jnp.int32),
        pltpu.VMEM((CHUNK,), jnp.float32),
        pltpu.VMEM((ROWS_PER_TILE,), jnp.float32),
        pltpu.VMEM_SHARED((N_P,), jnp.float32),
    ],
)
def _sc_degree(dst_hbm, out_hbm, idx_v, ones_v, buf_v, acc_sh):
    c = lax.axis_index("c")
    s = lax.axis_index("s")
    wid = s * NC + c

    def fill16(i, _):
        ones_v[pl.ds(i * 16, 16)] = jnp.ones((16,), jnp.float32)
        return 0
    lax.fori_loop(0, CHUNK // 16, fill16, 0)

    def zero16(i, _):
        buf_v[pl.ds(i * 16, 16)] = jnp.zeros((16,), jnp.float32)
        return 0
    lax.fori_loop(0, ROWS_PER_TILE // 16, zero16, 0)

    pltpu.sync_copy(buf_v, acc_sh.at[pl.ds(s * ROWS_PER_TILE, ROWS_PER_TILE)])
    plsc.subcore_barrier()

    pltpu.sync_copy(dst_hbm.at[pl.ds(wid * CHUNKS_PER_W, CHUNKS_PER_W)], idx_v)

    def body(j, _):
        pltpu.sync_copy(ones_v, acc_sh.at[idx_v.at[j]], add=True)
        return 0
    lax.fori_loop(0, CHUNKS_PER_W, body, 0)

    plsc.subcore_barrier()
    rows = pl.ds(s * ROWS_PER_TILE, ROWS_PER_TILE)
    pltpu.sync_copy(acc_sh.at[rows], buf_v)
    pltpu.sync_copy(buf_v, out_hbm.at[c, rows])


def _fused_body(x_ref, w1_ref, b1_ref, wg_ref, d0_ref, d1_ref,
                glo_ref, ghi_ref, dinv_ref):
    h = jnp.maximum(_mm_nt(x_ref[...], w1_ref[...]) + b1_ref[...], 0.0)
    hw = _mm_nt(h, wg_ref[...])
    deg = d0_ref[...] + d1_ref[...] + 1.0
    dinv = lax.rsqrt(deg)
    dinv_ref[...] = dinv
    g = hw * dinv
    gb = g.astype(jnp.bfloat16)
    glo_ref[...] = gb[:, :DH]
    ghi_ref[...] = gb[:, DH:]


def _tc_fused(xp, W1, b1_2d, Wg, deg0, deg1):
    return pl.pallas_call(
        _fused_body,
        grid=(N_P // BLK,),
        in_specs=[
            pl.BlockSpec((BLK, D), lambda i: (i, 0)),
            pl.BlockSpec((D, D), lambda i: (0, 0)),
            pl.BlockSpec((1, D), lambda i: (0, 0)),
            pl.BlockSpec((D, D), lambda i: (0, 0)),
            pl.BlockSpec((BLK, 1), lambda i: (i, 0)),
            pl.BlockSpec((BLK, 1), lambda i: (i, 0)),
        ],
        out_specs=[
            pl.BlockSpec((BLK, DH), lambda i: (i, 0)),
            pl.BlockSpec((BLK, DH), lambda i: (i, 0)),
            pl.BlockSpec((BLK, 1), lambda i: (i, 0)),
        ],
        out_shape=[
            jax.ShapeDtypeStruct((N_P, DH), jnp.bfloat16),
            jax.ShapeDtypeStruct((N_P, DH), jnp.bfloat16),
            jax.ShapeDtypeStruct((N_P, 1), jnp.float32),
        ],
    )(xp, W1, b1_2d, Wg, deg0, deg1)


@functools.partial(
    pl.kernel,
    out_type=[
        jax.ShapeDtypeStruct((N_P, DH), jnp.bfloat16),
        jax.ShapeDtypeStruct((N_P, DH), jnp.bfloat16),
    ],
    mesh=_mesh,
    scratch_types=[
        pltpu.VMEM((CHUNKS_PER_T, CHUNK), jnp.int32),
        pltpu.VMEM((CHUNKS_PER_T, CHUNK), jnp.int32),
        pltpu.VMEM((4, CHUNK, DH), jnp.bfloat16),
        pltpu.VMEM_SHARED((N_P, DH), jnp.bfloat16),
        [pltpu.SemaphoreType.DMA] * 4,
        [pltpu.SemaphoreType.DMA] * 4,
    ],
    compiler_params=pltpu.CompilerParams(use_tc_tiling_on_sc=False),
)
def _sc_scatter(src_hbm, dst_hbm, glo_hbm, ghi_hbm, outlo_hbm, outhi_hbm,
                src_v, dst_v, ring, acc_sh, gsems, ssems):
    c = lax.axis_index("c")
    s = lax.axis_index("s")

    def zero_row(i, _):
        ring[0, i // (DH // 32), pl.ds((i % (DH // 32)) * 32, 32)] = (
            jnp.zeros((32,), jnp.bfloat16))
        return 0
    lax.fori_loop(0, CHUNK * (DH // 32), zero_row, 0)

    def zero_acc(k, _):
        pltpu.sync_copy(ring.at[0],
                        acc_sh.at[pl.ds(s * ROWS_PER_TILE + k * CHUNK, CHUNK)])
        return 0
    lax.fori_loop(0, ROWS_PER_TILE // CHUNK, zero_acc, 0)
    plsc.subcore_barrier()

    pltpu.sync_copy(src_hbm.at[pl.ds(s * CHUNKS_PER_T, CHUNKS_PER_T)], src_v)
    pltpu.sync_copy(dst_hbm.at[pl.ds(s * CHUNKS_PER_T, CHUNKS_PER_T)], dst_v)

    def gather(g_hbm):
        for k in range(3):
            pltpu.async_copy(g_hbm.at[src_v.at[k]], ring.at[k], gsems[k])

        def body(jj, _):
            for k in range(4):
                j = jj * 4 + k
                m = (k + 3) % 4
                pltpu.make_async_copy(g_hbm.at[src_v.at[j]], ring.at[k],
                                      gsems[k]).wait()

                @pl.when(jnp.logical_and(j >= 1, j + 3 < CHUNKS_PER_T))
                def _():
                    pltpu.make_async_copy(ring.at[m],
                                          acc_sh.at[dst_v.at[j]],
                                          ssems[m]).wait()

                @pl.when(j + 3 < CHUNKS_PER_T)
                def _():
                    pltpu.async_copy(g_hbm.at[src_v.at[j + 3]], ring.at[m],
                                     gsems[m])

                pltpu.async_copy(ring.at[k], acc_sh.at[dst_v.at[j]],
                                 ssems[k], add=True)
            return 0
        lax.fori_loop(0, CHUNKS_PER_T // 4, body, 0)

        for k in range(4):
            pltpu.make_async_copy(ring.at[k], acc_sh.at[dst_v.at[0]],
                                  ssems[k]).wait()

    @pl.when(c == 0)
    def _():
        gather(glo_hbm)

    @pl.when(c == 1)
    def _():
        gather(ghi_hbm)

    plsc.subcore_barrier()

    def writeout(out_hbm):
        def wo(k, _):
            off = s * ROWS_PER_TILE + k * CHUNK
            pltpu.sync_copy(acc_sh.at[pl.ds(off, CHUNK)], ring.at[0])
            pltpu.sync_copy(ring.at[0], out_hbm.at[pl.ds(off, CHUNK)])
            return 0
        lax.fori_loop(0, ROWS_PER_TILE // CHUNK, wo, 0)

    @pl.when(c == 0)
    def _():
        writeout(outlo_hbm)

    @pl.when(c == 1)
    def _():
        writeout(outhi_hbm)


def _out_body(alo_ref, ahi_ref, glo_ref, ghi_ref, dinv_ref, bg_ref,
              w2_ref, b2_ref, o_ref):
    dinv = dinv_ref[...]
    alo = alo_ref[...].astype(jnp.float32)
    ahi = ahi_ref[...].astype(jnp.float32)
    glo = glo_ref[...].astype(jnp.float32)
    ghi = ghi_ref[...].astype(jnp.float32)
    conv = jnp.concatenate(
        [(alo + glo) * dinv,
         (ahi + ghi) * dinv], axis=1) + bg_ref[...]
    h2 = jnp.maximum(conv, 0.0)
    o_ref[...] = _mm_nt(h2, w2_ref[...]) + b2_ref[...]


def _tc_out(alo, ahi, glo, ghi, dinv, bg_2d, W2, b2_2d):
    return pl.pallas_call(
        _out_body,
        grid=(N_P // BLK,),
        in_specs=[
            pl.BlockSpec((BLK, DH), lambda i: (i, 0)),
            pl.BlockSpec((BLK, DH), lambda i: (i, 0)),
            pl.BlockSpec((BLK, DH), lambda i: (i, 0)),
            pl.BlockSpec((BLK, DH), lambda i: (i, 0)),
            pl.BlockSpec((BLK, 1), lambda i: (i, 0)),
            pl.BlockSpec((1, D), lambda i: (0, 0)),
            pl.BlockSpec((D, D), lambda i: (0, 0)),
            pl.BlockSpec((1, D), lambda i: (0, 0)),
        ],
        out_specs=pl.BlockSpec((BLK, D), lambda i: (i, 0)),
        out_shape=jax.ShapeDtypeStruct((N_P, D), jnp.float32),
    )(alo, ahi, glo, ghi, dinv, bg_2d, W2, b2_2d)


def kernel(x, edge_index, W1, b1, Wg, bg, W2, b2):
    ei = edge_index.astype(jnp.int32)
    n_pad_e = E_P - E_REAL
    pad_idx = N_REAL + (jnp.arange(n_pad_e, dtype=jnp.int32) % (N_P - N_REAL))
    srcp = jnp.concatenate([ei[0], pad_idx]).reshape(N_CHUNKS, CHUNK)
    dstp = jnp.concatenate([ei[1], pad_idx]).reshape(N_CHUNKS, CHUNK)

    xp = jnp.pad(x, ((0, N_P - N_REAL), (0, 0)))
    b1_2d = b1.reshape(1, D)
    bg_2d = bg.reshape(1, D)
    b2_2d = b2.reshape(1, D)

    deg_partials = _sc_degree(dstp)
    deg0 = deg_partials[0].reshape(N_P, 1)
    deg1 = deg_partials[1].reshape(N_P, 1)

    glo, ghi, dinv = _tc_fused(xp, W1, b1_2d, Wg, deg0, deg1)

    alo, ahi = _sc_scatter(srcp, dstp, glo, ghi)

    out = _tc_out(alo, ahi, glo, ghi, dinv, bg_2d, W2, b2_2d)
    return out[:N_REAL]


# speedup vs baseline: 1.7092x; 1.0634x over previous
import functools

import jax
import jax.numpy as jnp
from jax import lax
from jax.experimental import pallas as pl
from jax.experimental.pallas import tpu as pltpu
from jax.experimental.pallas import tpu_sc as plsc

N_REAL = 10000
D = 128
DH = D // 2
E_REAL = 320000

NC = 2
NS = 16
NW = NC * NS
CHUNK = 128
N_CHUNKS = 2560
E_P = N_CHUNKS * CHUNK
CHUNKS_PER_W = N_CHUNKS // NW
CHUNKS_PER_T = N_CHUNKS // NS
N_P = 10240
ROWS_PER_TILE = N_P // NS
BLK = 512

_mesh = plsc.VectorSubcoreMesh(core_axis_name="c", subcore_axis_name="s")


def _mm_nt(a, b):
    return lax.dot_general(a, b, (((1,), (1,)), ((), ())),
                           preferred_element_type=jnp.float32)


@functools.partial(
    pl.kernel,
    out_type=jax.ShapeDtypeStruct((NC, N_P), jnp.float32),
    mesh=_mesh,
    scratch_types=[
        pltpu.VMEM((CHUNKS_PER_W, CHUNK), jnp.int32),
        pltpu.VMEM((CHUNK,), jnp.float32),
        pltpu.VMEM((ROWS_PER_TILE,), jnp.float32),
        pltpu.VMEM_SHARED((N_P,), jnp.float32),
    ],
    compiler_params=pltpu.CompilerParams(use_tc_tiling_on_sc=False),
)
def _sc_degree(dst_hbm, out_hbm, idx_v, ones_v, buf_v, acc_sh):
    c = lax.axis_index("c")
    s = lax.axis_index("s")
    wid = s * NC + c

    def fill16(i, _):
        ones_v[pl.ds(i * 16, 16)] = jnp.ones((16,), jnp.float32)
        return 0
    lax.fori_loop(0, CHUNK // 16, fill16, 0)

    def zero16(i, _):
        buf_v[pl.ds(i * 16, 16)] = jnp.zeros((16,), jnp.float32)
        return 0
    lax.fori_loop(0, ROWS_PER_TILE // 16, zero16, 0)

    pltpu.sync_copy(buf_v, acc_sh.at[pl.ds(s * ROWS_PER_TILE, ROWS_PER_TILE)])
    plsc.subcore_barrier()

    pltpu.sync_copy(dst_hbm.at[pl.ds(wid * CHUNKS_PER_W, CHUNKS_PER_W)], idx_v)

    def body(j, _):
        pltpu.sync_copy(ones_v, acc_sh.at[idx_v.at[j]], add=True)
        return 0
    lax.fori_loop(0, CHUNKS_PER_W, body, 0)

    plsc.subcore_barrier()
    rows = pl.ds(s * ROWS_PER_TILE, ROWS_PER_TILE)
    pltpu.sync_copy(acc_sh.at[rows], buf_v)
    pltpu.sync_copy(buf_v, out_hbm.at[c, rows])


def _fused_body(x_ref, w1_ref, b1_ref, wg_ref, d0_ref, d1_ref,
                glo_ref, ghi_ref, dinv_ref):
    h = jnp.maximum(_mm_nt(x_ref[...], w1_ref[...]) + b1_ref[...], 0.0)
    hw = _mm_nt(h, wg_ref[...])
    deg = d0_ref[...] + d1_ref[...] + 1.0
    dinv = lax.rsqrt(deg)
    dinv_ref[...] = dinv
    g = hw * dinv
    gb = g.astype(jnp.bfloat16)
    glo_ref[...] = gb[:, :DH]
    ghi_ref[...] = gb[:, DH:]


def _tc_fused(xp, W1, b1_2d, Wg, deg0, deg1):
    return pl.pallas_call(
        _fused_body,
        grid=(N_P // BLK,),
        in_specs=[
            pl.BlockSpec((BLK, D), lambda i: (i, 0)),
            pl.BlockSpec((D, D), lambda i: (0, 0)),
            pl.BlockSpec((1, D), lambda i: (0, 0)),
            pl.BlockSpec((D, D), lambda i: (0, 0)),
            pl.BlockSpec((BLK, 1), lambda i: (i, 0)),
            pl.BlockSpec((BLK, 1), lambda i: (i, 0)),
        ],
        out_specs=[
            pl.BlockSpec((BLK, DH), lambda i: (i, 0)),
            pl.BlockSpec((BLK, DH), lambda i: (i, 0)),
            pl.BlockSpec((BLK, 1), lambda i: (i, 0)),
        ],
        out_shape=[
            jax.ShapeDtypeStruct((N_P, DH), jnp.bfloat16),
            jax.ShapeDtypeStruct((N_P, DH), jnp.bfloat16),
            jax.ShapeDtypeStruct((N_P, 1), jnp.float32),
        ],
    )(xp, W1, b1_2d, Wg, deg0, deg1)


@functools.partial(
    pl.kernel,
    out_type=[
        jax.ShapeDtypeStruct((N_P, DH), jnp.bfloat16),
        jax.ShapeDtypeStruct((N_P, DH), jnp.bfloat16),
    ],
    mesh=_mesh,
    scratch_types=[
        pltpu.VMEM((CHUNKS_PER_T, CHUNK), jnp.int32),
        pltpu.VMEM((CHUNKS_PER_T, CHUNK), jnp.int32),
        pltpu.VMEM((4, CHUNK, DH), jnp.bfloat16),
        pltpu.VMEM_SHARED((N_P, DH), jnp.bfloat16),
        [pltpu.SemaphoreType.DMA] * 4,
        [pltpu.SemaphoreType.DMA] * 4,
    ],
    compiler_params=pltpu.CompilerParams(use_tc_tiling_on_sc=False),
)
def _sc_scatter(src_hbm, dst_hbm, glo_hbm, ghi_hbm, outlo_hbm, outhi_hbm,
                src_v, dst_v, ring, acc_sh, gsems, ssems):
    c = lax.axis_index("c")
    s = lax.axis_index("s")

    def zero_row(i, _):
        ring[0, i // (DH // 32), pl.ds((i % (DH // 32)) * 32, 32)] = (
            jnp.zeros((32,), jnp.bfloat16))
        return 0
    lax.fori_loop(0, CHUNK * (DH // 32), zero_row, 0)

    def zero_acc(k, _):
        pltpu.sync_copy(ring.at[0],
                        acc_sh.at[pl.ds(s * ROWS_PER_TILE + k * CHUNK, CHUNK)])
        return 0
    lax.fori_loop(0, ROWS_PER_TILE // CHUNK, zero_acc, 0)
    plsc.subcore_barrier()

    pltpu.sync_copy(src_hbm.at[pl.ds(s * CHUNKS_PER_T, CHUNKS_PER_T)], src_v)
    pltpu.sync_copy(dst_hbm.at[pl.ds(s * CHUNKS_PER_T, CHUNKS_PER_T)], dst_v)

    def gather(g_hbm):
        for k in range(3):
            pltpu.async_copy(g_hbm.at[src_v.at[k]], ring.at[k], gsems[k])

        def body(jj, _):
            for k in range(4):
                j = jj * 4 + k
                m = (k + 3) % 4
                pltpu.make_async_copy(g_hbm.at[src_v.at[j]], ring.at[k],
                                      gsems[k]).wait()

                @pl.when(jnp.logical_and(j >= 1, j + 3 < CHUNKS_PER_T))
                def _():
                    pltpu.make_async_copy(ring.at[m],
                                          acc_sh.at[dst_v.at[j]],
                                          ssems[m]).wait()

                @pl.when(j + 3 < CHUNKS_PER_T)
                def _():
                    pltpu.async_copy(g_hbm.at[src_v.at[j + 3]], ring.at[m],
                                     gsems[m])

                pltpu.async_copy(ring.at[k], acc_sh.at[dst_v.at[j]],
                                 ssems[k], add=True)
            return 0
        lax.fori_loop(0, CHUNKS_PER_T // 4, body, 0)

        for k in range(4):
            pltpu.make_async_copy(ring.at[k], acc_sh.at[dst_v.at[0]],
                                  ssems[k]).wait()

    @pl.when(c == 0)
    def _():
        gather(glo_hbm)

    @pl.when(c == 1)
    def _():
        gather(ghi_hbm)

    plsc.subcore_barrier()

    def writeout(out_hbm):
        def wo(k, _):
            off = s * ROWS_PER_TILE + k * CHUNK
            pltpu.sync_copy(acc_sh.at[pl.ds(off, CHUNK)], ring.at[0])
            pltpu.sync_copy(ring.at[0], out_hbm.at[pl.ds(off, CHUNK)])
            return 0
        lax.fori_loop(0, ROWS_PER_TILE // CHUNK, wo, 0)

    @pl.when(c == 0)
    def _():
        writeout(outlo_hbm)

    @pl.when(c == 1)
    def _():
        writeout(outhi_hbm)


def _out_body(alo_ref, ahi_ref, glo_ref, ghi_ref, dinv_ref, bg_ref,
              w2_ref, b2_ref, o_ref):
    dinv = dinv_ref[...]
    alo = alo_ref[...].astype(jnp.float32)
    ahi = ahi_ref[...].astype(jnp.float32)
    glo = glo_ref[...].astype(jnp.float32)
    ghi = ghi_ref[...].astype(jnp.float32)
    conv = jnp.concatenate(
        [(alo + glo) * dinv,
         (ahi + ghi) * dinv], axis=1) + bg_ref[...]
    h2 = jnp.maximum(conv, 0.0)
    o_ref[...] = _mm_nt(h2, w2_ref[...]) + b2_ref[...]


OBLK = 1000


def _tc_out(alo, ahi, glo, ghi, dinv, bg_2d, W2, b2_2d):
    return pl.pallas_call(
        _out_body,
        grid=(N_REAL // OBLK,),
        in_specs=[
            pl.BlockSpec((OBLK, DH), lambda i: (i, 0)),
            pl.BlockSpec((OBLK, DH), lambda i: (i, 0)),
            pl.BlockSpec((OBLK, DH), lambda i: (i, 0)),
            pl.BlockSpec((OBLK, DH), lambda i: (i, 0)),
            pl.BlockSpec((OBLK, 1), lambda i: (i, 0)),
            pl.BlockSpec((1, D), lambda i: (0, 0)),
            pl.BlockSpec((D, D), lambda i: (0, 0)),
            pl.BlockSpec((1, D), lambda i: (0, 0)),
        ],
        out_specs=pl.BlockSpec((OBLK, D), lambda i: (i, 0)),
        out_shape=jax.ShapeDtypeStruct((N_REAL, D), jnp.float32),
    )(alo, ahi, glo, ghi, dinv, bg_2d, W2, b2_2d)


def kernel(x, edge_index, W1, b1, Wg, bg, W2, b2):
    ei = edge_index.astype(jnp.int32)
    n_pad_e = E_P - E_REAL
    pad_idx = N_REAL + (jnp.arange(n_pad_e, dtype=jnp.int32) % (N_P - N_REAL))
    srcp = jnp.concatenate([ei[0], pad_idx]).reshape(N_CHUNKS, CHUNK)
    dstp = jnp.concatenate([ei[1], pad_idx]).reshape(N_CHUNKS, CHUNK)

    b1_2d = b1.reshape(1, D)
    bg_2d = bg.reshape(1, D)
    b2_2d = b2.reshape(1, D)

    deg_partials = _sc_degree(dstp)
    deg0 = deg_partials[0].reshape(N_P, 1)
    deg1 = deg_partials[1].reshape(N_P, 1)

    glo, ghi, dinv = _tc_fused(x, W1, b1_2d, Wg, deg0, deg1)

    alo, ahi = _sc_scatter(srcp, dstp, glo, ghi)

    return _tc_out(alo, ahi, glo, ghi, dinv, bg_2d, W2, b2_2d)


# speedup vs baseline: 1.8186x; 1.0640x over previous
import functools

import jax
import jax.numpy as jnp
from jax import lax
from jax.experimental import pallas as pl
from jax.experimental.pallas import tpu as pltpu
from jax.experimental.pallas import tpu_sc as plsc

N_REAL = 10000
D = 128
DH = D // 2
E_REAL = 320000

NC = 2
NS = 16
NW = NC * NS
CHUNK = 128
N_CHUNKS = E_REAL // CHUNK
DEG_BASE = N_CHUNKS // NW
DEG_EXTRA = N_CHUNKS - DEG_BASE * NW
SCAT_BASE = N_CHUNKS // NS
SCAT_EXTRA = N_CHUNKS - SCAT_BASE * NS
N_P = 10240
ROWS_PER_TILE = N_P // NS
BLK = 512

_mesh = plsc.VectorSubcoreMesh(core_axis_name="c", subcore_axis_name="s")


def _mm_nt(a, b):
    return lax.dot_general(a, b, (((1,), (1,)), ((), ())),
                           preferred_element_type=jnp.float32)


@functools.partial(
    pl.kernel,
    out_type=jax.ShapeDtypeStruct((NC, N_P), jnp.float32),
    mesh=_mesh,
    scratch_types=[
        pltpu.VMEM((DEG_BASE + 1, CHUNK), jnp.int32),
        pltpu.VMEM((CHUNK,), jnp.float32),
        pltpu.VMEM((ROWS_PER_TILE,), jnp.float32),
        pltpu.VMEM_SHARED((N_P,), jnp.float32),
    ],
    compiler_params=pltpu.CompilerParams(use_tc_tiling_on_sc=False),
)
def _sc_degree(ei_hbm, out_hbm, idx_v, ones_v, buf_v, acc_sh):
    c = lax.axis_index("c")
    s = lax.axis_index("s")
    wid = s * NC + c
    start = wid * DEG_BASE + jnp.minimum(wid, DEG_EXTRA)

    def fill16(i, _):
        ones_v[pl.ds(i * 16, 16)] = jnp.ones((16,), jnp.float32)
        return 0
    lax.fori_loop(0, CHUNK // 16, fill16, 0)

    def zero16(i, _):
        buf_v[pl.ds(i * 16, 16)] = jnp.zeros((16,), jnp.float32)
        return 0
    lax.fori_loop(0, ROWS_PER_TILE // 16, zero16, 0)

    pltpu.sync_copy(buf_v, acc_sh.at[pl.ds(s * ROWS_PER_TILE, ROWS_PER_TILE)])
    plsc.subcore_barrier()

    pltpu.sync_copy(ei_hbm.at[1, pl.ds(start, DEG_BASE)],
                    idx_v.at[pl.ds(0, DEG_BASE)])

    @pl.when(wid < DEG_EXTRA)
    def _():
        pltpu.sync_copy(ei_hbm.at[1, start + DEG_BASE], idx_v.at[DEG_BASE])

    def body(j, _):
        pltpu.sync_copy(ones_v, acc_sh.at[idx_v.at[j]], add=True)
        return 0
    lax.fori_loop(0, DEG_BASE, body, 0)

    @pl.when(wid < DEG_EXTRA)
    def _():
        pltpu.sync_copy(ones_v, acc_sh.at[idx_v.at[DEG_BASE]], add=True)

    plsc.subcore_barrier()
    rows = pl.ds(s * ROWS_PER_TILE, ROWS_PER_TILE)
    pltpu.sync_copy(acc_sh.at[rows], buf_v)
    pltpu.sync_copy(buf_v, out_hbm.at[c, rows])


def _fused_body(x_ref, w1_ref, b1_ref, wg_ref, d0_ref, d1_ref,
                glo_ref, ghi_ref, dinv_ref):
    h = jnp.maximum(_mm_nt(x_ref[...], w1_ref[...]) + b1_ref[...], 0.0)
    hw = _mm_nt(h, wg_ref[...])
    deg = d0_ref[...] + d1_ref[...] + 1.0
    dinv = lax.rsqrt(deg)
    dinv_ref[...] = dinv
    g = hw * dinv
    gb = g.astype(jnp.bfloat16)
    glo_ref[...] = gb[:, :DH]
    ghi_ref[...] = gb[:, DH:]


def _tc_fused(xp, W1, b1_2d, Wg, deg0, deg1):
    return pl.pallas_call(
        _fused_body,
        grid=(N_P // BLK,),
        in_specs=[
            pl.BlockSpec((BLK, D), lambda i: (i, 0)),
            pl.BlockSpec((D, D), lambda i: (0, 0)),
            pl.BlockSpec((1, D), lambda i: (0, 0)),
            pl.BlockSpec((D, D), lambda i: (0, 0)),
            pl.BlockSpec((BLK, 1), lambda i: (i, 0)),
            pl.BlockSpec((BLK, 1), lambda i: (i, 0)),
        ],
        out_specs=[
            pl.BlockSpec((BLK, DH), lambda i: (i, 0)),
            pl.BlockSpec((BLK, DH), lambda i: (i, 0)),
            pl.BlockSpec((BLK, 1), lambda i: (i, 0)),
        ],
        out_shape=[
            jax.ShapeDtypeStruct((N_P, DH), jnp.bfloat16),
            jax.ShapeDtypeStruct((N_P, DH), jnp.bfloat16),
            jax.ShapeDtypeStruct((N_P, 1), jnp.float32),
        ],
    )(xp, W1, b1_2d, Wg, deg0, deg1)


@functools.partial(
    pl.kernel,
    out_type=[
        jax.ShapeDtypeStruct((N_P, DH), jnp.bfloat16),
        jax.ShapeDtypeStruct((N_P, DH), jnp.bfloat16),
    ],
    mesh=_mesh,
    scratch_types=[
        pltpu.VMEM((SCAT_BASE + 1, CHUNK), jnp.int32),
        pltpu.VMEM((SCAT_BASE + 1, CHUNK), jnp.int32),
        pltpu.VMEM((4, CHUNK, DH), jnp.bfloat16),
        pltpu.VMEM_SHARED((N_P, DH), jnp.bfloat16),
        [pltpu.SemaphoreType.DMA] * 4,
        [pltpu.SemaphoreType.DMA] * 4,
    ],
    compiler_params=pltpu.CompilerParams(use_tc_tiling_on_sc=False),
)
def _sc_scatter(ei_hbm, glo_hbm, ghi_hbm, outlo_hbm, outhi_hbm,
                src_v, dst_v, ring, acc_sh, gsems, ssems):
    c = lax.axis_index("c")
    s = lax.axis_index("s")
    start = s * SCAT_BASE + jnp.minimum(s, SCAT_EXTRA)
    cnt = SCAT_BASE + jnp.where(s < SCAT_EXTRA, 1, 0)

    def zero_row(i, _):
        ring[0, i // (DH // 32), pl.ds((i % (DH // 32)) * 32, 32)] = (
            jnp.zeros((32,), jnp.bfloat16))
        return 0
    lax.fori_loop(0, CHUNK * (DH // 32), zero_row, 0)

    def zero_acc(k, _):
        pltpu.sync_copy(ring.at[0],
                        acc_sh.at[pl.ds(s * ROWS_PER_TILE + k * CHUNK, CHUNK)])
        return 0
    lax.fori_loop(0, ROWS_PER_TILE // CHUNK, zero_acc, 0)
    plsc.subcore_barrier()

    pltpu.sync_copy(ei_hbm.at[0, pl.ds(start, SCAT_BASE)],
                    src_v.at[pl.ds(0, SCAT_BASE)])
    pltpu.sync_copy(ei_hbm.at[1, pl.ds(start, SCAT_BASE)],
                    dst_v.at[pl.ds(0, SCAT_BASE)])

    @pl.when(s < SCAT_EXTRA)
    def _():
        pltpu.sync_copy(ei_hbm.at[0, start + SCAT_BASE], src_v.at[SCAT_BASE])
        pltpu.sync_copy(ei_hbm.at[1, start + SCAT_BASE], dst_v.at[SCAT_BASE])

    def gather(g_hbm):
        for k in range(3):
            pltpu.async_copy(g_hbm.at[src_v.at[k]], ring.at[k], gsems[k])

        def body(jj, _):
            for k in range(4):
                j = jj * 4 + k
                m = (k + 3) % 4
                pltpu.make_async_copy(g_hbm.at[src_v.at[j]], ring.at[k],
                                      gsems[k]).wait()

                @pl.when(jnp.logical_and(j >= 1, j + 3 < cnt))
                def _():
                    pltpu.make_async_copy(ring.at[m],
                                          acc_sh.at[dst_v.at[j]],
                                          ssems[m]).wait()

                @pl.when(j + 3 < cnt)
                def _():
                    pltpu.async_copy(g_hbm.at[src_v.at[j + 3]], ring.at[m],
                                     gsems[m])

                pltpu.async_copy(ring.at[k], acc_sh.at[dst_v.at[j]],
                                 ssems[k], add=True)
            return 0
        lax.fori_loop(0, SCAT_BASE // 4, body, 0)

        @pl.when(cnt > SCAT_BASE)
        def _():
            pltpu.make_async_copy(g_hbm.at[src_v.at[SCAT_BASE]], ring.at[0],
                                  gsems[0]).wait()
            pltpu.async_copy(ring.at[0], acc_sh.at[dst_v.at[SCAT_BASE]],
                             ssems[0], add=True)

        for k in range(4):
            pltpu.make_async_copy(ring.at[k], acc_sh.at[dst_v.at[0]],
                                  ssems[k]).wait()

    @pl.when(c == 0)
    def _():
        gather(glo_hbm)

    @pl.when(c == 1)
    def _():
        gather(ghi_hbm)

    plsc.subcore_barrier()

    def writeout(out_hbm):
        def wo(k, _):
            off = s * ROWS_PER_TILE + k * CHUNK
            pltpu.sync_copy(acc_sh.at[pl.ds(off, CHUNK)], ring.at[0])
            pltpu.sync_copy(ring.at[0], out_hbm.at[pl.ds(off, CHUNK)])
            return 0
        lax.fori_loop(0, ROWS_PER_TILE // CHUNK, wo, 0)

    @pl.when(c == 0)
    def _():
        writeout(outlo_hbm)

    @pl.when(c == 1)
    def _():
        writeout(outhi_hbm)


def _out_body(alo_ref, ahi_ref, glo_ref, ghi_ref, dinv_ref, bg_ref,
              w2_ref, b2_ref, o_ref):
    dinv = dinv_ref[...]
    alo = alo_ref[...].astype(jnp.float32)
    ahi = ahi_ref[...].astype(jnp.float32)
    glo = glo_ref[...].astype(jnp.float32)
    ghi = ghi_ref[...].astype(jnp.float32)
    conv = jnp.concatenate(
        [(alo + glo) * dinv,
         (ahi + ghi) * dinv], axis=1) + bg_ref[...]
    h2 = jnp.maximum(conv, 0.0)
    o_ref[...] = _mm_nt(h2, w2_ref[...]) + b2_ref[...]


OBLK = 1000


def _tc_out(alo, ahi, glo, ghi, dinv, bg_2d, W2, b2_2d):
    return pl.pallas_call(
        _out_body,
        grid=(N_REAL // OBLK,),
        in_specs=[
            pl.BlockSpec((OBLK, DH), lambda i: (i, 0)),
            pl.BlockSpec((OBLK, DH), lambda i: (i, 0)),
            pl.BlockSpec((OBLK, DH), lambda i: (i, 0)),
            pl.BlockSpec((OBLK, DH), lambda i: (i, 0)),
            pl.BlockSpec((OBLK, 1), lambda i: (i, 0)),
            pl.BlockSpec((1, D), lambda i: (0, 0)),
            pl.BlockSpec((D, D), lambda i: (0, 0)),
            pl.BlockSpec((1, D), lambda i: (0, 0)),
        ],
        out_specs=pl.BlockSpec((OBLK, D), lambda i: (i, 0)),
        out_shape=jax.ShapeDtypeStruct((N_REAL, D), jnp.float32),
    )(alo, ahi, glo, ghi, dinv, bg_2d, W2, b2_2d)


def kernel(x, edge_index, W1, b1, Wg, bg, W2, b2):
    ei3 = edge_index.astype(jnp.int32).reshape(2, N_CHUNKS, CHUNK)

    b1_2d = b1.reshape(1, D)
    bg_2d = bg.reshape(1, D)
    b2_2d = b2.reshape(1, D)

    deg_partials = _sc_degree(ei3)
    deg0 = deg_partials[0].reshape(N_P, 1)
    deg1 = deg_partials[1].reshape(N_P, 1)

    glo, ghi, dinv = _tc_fused(x, W1, b1_2d, Wg, deg0, deg1)

    alo, ahi = _sc_scatter(ei3, glo, ghi)

    return _tc_out(alo, ahi, glo, ghi, dinv, bg_2d, W2, b2_2d)


# speedup vs baseline: 2.0499x; 1.1272x over previous
import functools

import jax
import jax.numpy as jnp
from jax import lax
from jax.experimental import pallas as pl
from jax.experimental.pallas import tpu as pltpu
from jax.experimental.pallas import tpu_sc as plsc

N_REAL = 10000
D = 128
DH = D // 2
E_REAL = 320000

NC = 2
NS = 16
NW = NC * NS
CHUNK = 128
N_CHUNKS = E_REAL // CHUNK
DEG_BASE = N_CHUNKS // NW
DEG_EXTRA = N_CHUNKS - DEG_BASE * NW
SCAT_BASE = N_CHUNKS // NS
SCAT_EXTRA = N_CHUNKS - SCAT_BASE * NS
N_P = 10240
ROWS_PER_TILE = N_P // NS
BLK = 1024
NBUF = 6

_mesh = plsc.VectorSubcoreMesh(core_axis_name="c", subcore_axis_name="s")


def _mm_nt(a, b):
    return lax.dot_general(a, b, (((1,), (1,)), ((), ())),
                           preferred_element_type=jnp.float32)


@functools.partial(
    pl.kernel,
    out_type=jax.ShapeDtypeStruct((NC, N_P), jnp.float32),
    mesh=_mesh,
    scratch_types=[
        pltpu.VMEM((DEG_BASE + 1, CHUNK), jnp.int32),
        pltpu.VMEM((CHUNK,), jnp.float32),
        pltpu.VMEM((ROWS_PER_TILE,), jnp.float32),
        pltpu.VMEM_SHARED((N_P,), jnp.float32),
    ],
    compiler_params=pltpu.CompilerParams(use_tc_tiling_on_sc=False),
)
def _sc_degree(ei_hbm, out_hbm, idx_v, ones_v, buf_v, acc_sh):
    c = lax.axis_index("c")
    s = lax.axis_index("s")
    wid = s * NC + c
    start = wid * DEG_BASE + jnp.minimum(wid, DEG_EXTRA)

    def fill16(i, _):
        ones_v[pl.ds(i * 16, 16)] = jnp.ones((16,), jnp.float32)
        return 0
    lax.fori_loop(0, CHUNK // 16, fill16, 0)

    def zero16(i, _):
        buf_v[pl.ds(i * 16, 16)] = jnp.zeros((16,), jnp.float32)
        return 0
    lax.fori_loop(0, ROWS_PER_TILE // 16, zero16, 0)

    pltpu.sync_copy(buf_v, acc_sh.at[pl.ds(s * ROWS_PER_TILE, ROWS_PER_TILE)])
    plsc.subcore_barrier()

    pltpu.sync_copy(ei_hbm.at[1, pl.ds(start, DEG_BASE)],
                    idx_v.at[pl.ds(0, DEG_BASE)])

    @pl.when(wid < DEG_EXTRA)
    def _():
        pltpu.sync_copy(ei_hbm.at[1, start + DEG_BASE], idx_v.at[DEG_BASE])

    def body(j, _):
        pltpu.sync_copy(ones_v, acc_sh.at[idx_v.at[j]], add=True)
        return 0
    lax.fori_loop(0, DEG_BASE, body, 0)

    @pl.when(wid < DEG_EXTRA)
    def _():
        pltpu.sync_copy(ones_v, acc_sh.at[idx_v.at[DEG_BASE]], add=True)

    plsc.subcore_barrier()
    rows = pl.ds(s * ROWS_PER_TILE, ROWS_PER_TILE)
    pltpu.sync_copy(acc_sh.at[rows], buf_v)
    pltpu.sync_copy(buf_v, out_hbm.at[c, rows])


def _fused_body(x_ref, w1_ref, b1_ref, wg_ref, d0_ref, d1_ref,
                glo_ref, ghi_ref, dinv_ref):
    h = jnp.maximum(_mm_nt(x_ref[...], w1_ref[...]) + b1_ref[...], 0.0)
    hw = _mm_nt(h, wg_ref[...])
    deg = d0_ref[...] + d1_ref[...] + 1.0
    dinv = lax.rsqrt(deg)
    dinv_ref[...] = dinv
    g = hw * dinv
    gb = g.astype(jnp.bfloat16)
    glo_ref[...] = gb[:, :DH]
    ghi_ref[...] = gb[:, DH:]


def _tc_fused(xp, W1, b1_2d, Wg, deg0, deg1):
    return pl.pallas_call(
        _fused_body,
        grid=(N_P // BLK,),
        in_specs=[
            pl.BlockSpec((BLK, D), lambda i: (i, 0)),
            pl.BlockSpec((D, D), lambda i: (0, 0)),
            pl.BlockSpec((1, D), lambda i: (0, 0)),
            pl.BlockSpec((D, D), lambda i: (0, 0)),
            pl.BlockSpec((BLK, 1), lambda i: (i, 0)),
            pl.BlockSpec((BLK, 1), lambda i: (i, 0)),
        ],
        out_specs=[
            pl.BlockSpec((BLK, DH), lambda i: (i, 0)),
            pl.BlockSpec((BLK, DH), lambda i: (i, 0)),
            pl.BlockSpec((BLK, 1), lambda i: (i, 0)),
        ],
        out_shape=[
            jax.ShapeDtypeStruct((N_P, DH), jnp.bfloat16),
            jax.ShapeDtypeStruct((N_P, DH), jnp.bfloat16),
            jax.ShapeDtypeStruct((N_P, 1), jnp.float32),
        ],
    )(xp, W1, b1_2d, Wg, deg0, deg1)


@functools.partial(
    pl.kernel,
    out_type=[
        jax.ShapeDtypeStruct((N_P, DH), jnp.bfloat16),
        jax.ShapeDtypeStruct((N_P, DH), jnp.bfloat16),
    ],
    mesh=_mesh,
    scratch_types=[
        pltpu.VMEM((SCAT_BASE + 1, CHUNK), jnp.int32),
        pltpu.VMEM((SCAT_BASE + 1, CHUNK), jnp.int32),
        pltpu.VMEM((NBUF, CHUNK, DH), jnp.bfloat16),
        pltpu.VMEM_SHARED((N_P, DH), jnp.bfloat16),
        [pltpu.SemaphoreType.DMA] * NBUF,
        [pltpu.SemaphoreType.DMA] * NBUF,
    ],
    compiler_params=pltpu.CompilerParams(use_tc_tiling_on_sc=False),
)
def _sc_scatter(ei_hbm, glo_hbm, ghi_hbm, outlo_hbm, outhi_hbm,
                src_v, dst_v, ring, acc_sh, gsems, ssems):
    c = lax.axis_index("c")
    s = lax.axis_index("s")
    start = s * SCAT_BASE + jnp.minimum(s, SCAT_EXTRA)
    cnt = SCAT_BASE + jnp.where(s < SCAT_EXTRA, 1, 0)

    def zero_row(i, _):
        ring[0, i // (DH // 32), pl.ds((i % (DH // 32)) * 32, 32)] = (
            jnp.zeros((32,), jnp.bfloat16))
        return 0
    lax.fori_loop(0, CHUNK * (DH // 32), zero_row, 0)

    def zero_acc(k, _):
        pltpu.sync_copy(ring.at[0],
                        acc_sh.at[pl.ds(s * ROWS_PER_TILE + k * CHUNK, CHUNK)])
        return 0
    lax.fori_loop(0, ROWS_PER_TILE // CHUNK, zero_acc, 0)
    plsc.subcore_barrier()

    pltpu.sync_copy(ei_hbm.at[0, pl.ds(start, SCAT_BASE)],
                    src_v.at[pl.ds(0, SCAT_BASE)])
    pltpu.sync_copy(ei_hbm.at[1, pl.ds(start, SCAT_BASE)],
                    dst_v.at[pl.ds(0, SCAT_BASE)])

    @pl.when(s < SCAT_EXTRA)
    def _():
        pltpu.sync_copy(ei_hbm.at[0, start + SCAT_BASE], src_v.at[SCAT_BASE])
        pltpu.sync_copy(ei_hbm.at[1, start + SCAT_BASE], dst_v.at[SCAT_BASE])

    def gather(g_hbm):
        for k in range(NBUF - 1):
            pltpu.async_copy(g_hbm.at[src_v.at[k]], ring.at[k], gsems[k])

        def body(jj, _):
            for k in range(NBUF):
                j = jj * NBUF + k
                m = (k + NBUF - 1) % NBUF
                pltpu.make_async_copy(g_hbm.at[src_v.at[j]], ring.at[k],
                                      gsems[k]).wait()

                @pl.when(jnp.logical_and(j >= 1, j + NBUF - 1 < cnt))
                def _():
                    pltpu.make_async_copy(ring.at[m],
                                          acc_sh.at[dst_v.at[j]],
                                          ssems[m]).wait()

                @pl.when(j + NBUF - 1 < cnt)
                def _():
                    pltpu.async_copy(g_hbm.at[src_v.at[j + NBUF - 1]],
                                     ring.at[m], gsems[m])

                pltpu.async_copy(ring.at[k], acc_sh.at[dst_v.at[j]],
                                 ssems[k], add=True)
            return 0
        lax.fori_loop(0, SCAT_BASE // NBUF, body, 0)

        @pl.when(cnt > SCAT_BASE)
        def _():
            pltpu.make_async_copy(g_hbm.at[src_v.at[SCAT_BASE]], ring.at[0],
                                  gsems[0]).wait()
            pltpu.async_copy(ring.at[0], acc_sh.at[dst_v.at[SCAT_BASE]],
                             ssems[0], add=True)

        for k in range(NBUF):
            pltpu.make_async_copy(ring.at[k], acc_sh.at[dst_v.at[0]],
                                  ssems[k]).wait()

    @pl.when(c == 0)
    def _():
        gather(glo_hbm)

    @pl.when(c == 1)
    def _():
        gather(ghi_hbm)

    plsc.subcore_barrier()

    def writeout(out_hbm):
        def wo(k, _):
            off = s * ROWS_PER_TILE + k * CHUNK
            pltpu.sync_copy(acc_sh.at[pl.ds(off, CHUNK)], ring.at[0])
            pltpu.sync_copy(ring.at[0], out_hbm.at[pl.ds(off, CHUNK)])
            return 0
        lax.fori_loop(0, ROWS_PER_TILE // CHUNK, wo, 0)

    @pl.when(c == 0)
    def _():
        writeout(outlo_hbm)

    @pl.when(c == 1)
    def _():
        writeout(outhi_hbm)


def _out_body(alo_ref, ahi_ref, glo_ref, ghi_ref, dinv_ref, bg_ref,
              w2_ref, b2_ref, o_ref):
    dinv = dinv_ref[...]
    alo = alo_ref[...].astype(jnp.float32)
    ahi = ahi_ref[...].astype(jnp.float32)
    glo = glo_ref[...].astype(jnp.float32)
    ghi = ghi_ref[...].astype(jnp.float32)
    conv = jnp.concatenate(
        [(alo + glo) * dinv,
         (ahi + ghi) * dinv], axis=1) + bg_ref[...]
    h2 = jnp.maximum(conv, 0.0)
    o_ref[...] = _mm_nt(h2, w2_ref[...]) + b2_ref[...]


OBLK = 2000


def _tc_out(alo, ahi, glo, ghi, dinv, bg_2d, W2, b2_2d):
    return pl.pallas_call(
        _out_body,
        grid=(N_REAL // OBLK,),
        in_specs=[
            pl.BlockSpec((OBLK, DH), lambda i: (i, 0)),
            pl.BlockSpec((OBLK, DH), lambda i: (i, 0)),
            pl.BlockSpec((OBLK, DH), lambda i: (i, 0)),
            pl.BlockSpec((OBLK, DH), lambda i: (i, 0)),
            pl.BlockSpec((OBLK, 1), lambda i: (i, 0)),
            pl.BlockSpec((1, D), lambda i: (0, 0)),
            pl.BlockSpec((D, D), lambda i: (0, 0)),
            pl.BlockSpec((1, D), lambda i: (0, 0)),
        ],
        out_specs=pl.BlockSpec((OBLK, D), lambda i: (i, 0)),
        out_shape=jax.ShapeDtypeStruct((N_REAL, D), jnp.float32),
    )(alo, ahi, glo, ghi, dinv, bg_2d, W2, b2_2d)


def kernel(x, edge_index, W1, b1, Wg, bg, W2, b2):
    ei3 = edge_index.astype(jnp.int32).reshape(2, N_CHUNKS, CHUNK)

    b1_2d = b1.reshape(1, D)
    bg_2d = bg.reshape(1, D)
    b2_2d = b2.reshape(1, D)

    deg_partials = _sc_degree(ei3)
    deg0 = deg_partials[0].reshape(N_P, 1)
    deg1 = deg_partials[1].reshape(N_P, 1)

    glo, ghi, dinv = _tc_fused(x, W1, b1_2d, Wg, deg0, deg1)

    alo, ahi = _sc_scatter(ei3, glo, ghi)

    return _tc_out(alo, ahi, glo, ghi, dinv, bg_2d, W2, b2_2d)
